# baseline (speedup 1.0000x reference)
import jax
import jax.numpy as jnp
from jax import lax
from jax.experimental import pallas as pl
from jax.experimental.pallas import tpu as pltpu
from jax.experimental.pallas import tpu_sc as plsc

N_U = 25000
N_I = 25000
N = N_U + N_I
E = 800000
D = 64
H = 32
LAYERS = 3
NC = 2
NS = 16
LN = 16

EPT = E // NS
CH = 400
NCHUNK = EPT // CH
NPAIR = (NCHUNK - 1) // 2
SUB = 80
NSUB = CH // SUB
VPS = SUB // LN

RQ = 3128
RQ_LAST = N - (NS - 1) * RQ
NZ = RQ // CH
MCH = 80
NMCH = N // MCH

F32 = jnp.float32
I32 = jnp.int32


def _sc_body(ego0, eidx, wb, out, lb, acc,
             rowsA, rowsB, srcvA, srcvB, wvA, wvB, didxA, didxB,
             oidx, seml, semd, semgA, semgB, sems):
  c = lax.axis_index("c")
  s = lax.axis_index("s")
  ebase0 = s * EPT
  rbase = s * RQ
  is_last = s == NS - 1
  zeros = jnp.zeros((LN,), F32)

  bufs_a = (rowsA, srcvA, wvA, didxA, semgA)
  bufs_b = (rowsB, srcvB, wvB, didxB, semgB)

  def _loads_sw(bufs, ci):
    _, srcv, wv, _, _ = bufs
    eoff = ebase0 + ci * CH
    pltpu.async_copy(eidx.at[0, pl.ds(eoff, CH)], srcv, seml)
    pltpu.async_copy(wb.at[pl.ds(eoff, CH)], wv, seml)

  def _drain_loads_sw(bufs):
    _, srcv, wv, _, _ = bufs
    pltpu.make_async_copy(eidx.at[0, pl.ds(0, CH)], srcv, seml).wait()
    pltpu.make_async_copy(wb.at[pl.ds(0, CH)], wv, seml).wait()

  def _fire_didx(bufs, ci):
    _, _, _, didx, _ = bufs
    eoff = ebase0 + ci * CH
    for j in range(NSUB):
      pltpu.async_copy(eidx.at[1, pl.ds(eoff + j * SUB, SUB)],
                       didx.at[j], semd)

  def _drain_didx(bufs):
    _, _, _, didx, _ = bufs
    for j in range(NSUB):
      pltpu.make_async_copy(eidx.at[1, pl.ds(0, SUB)], didx.at[j],
                            semd).wait()

  def _transform(bufs):
    _, srcv, _, _, _ = bufs

    def _tf(k, _):
      k16 = pl.multiple_of(k * LN, LN)
      sv = srcv[pl.ds(k16, LN)]
      srcv[pl.ds(k16, LN)] = 2 * sv + c
      return _
    lax.fori_loop(0, CH // LN, _tf, None)

  def _fire_gathers(bufs, table):
    rows, srcv, _, _, semg = bufs
    pltpu.async_copy(table.at[srcv], rows, semg)

  def _drain_gathers(bufs, table):
    rows, srcv, _, _, semg = bufs
    pltpu.make_async_copy(table.at[srcv], rows, semg).wait()

  def _multiply(bufs):
    rows, _, wv, _, _ = bufs

    def _mul(g, _):
      goff = pl.multiple_of(g * LN, LN)
      w16 = wv[pl.ds(goff, LN)]
      for t in range(LN):
        wbc = w16.at[jnp.full((LN,), t, I32)].get(mode="promise_in_bounds")
        e = goff + t
        rows[e, pl.ds(0, LN)] = rows[e, pl.ds(0, LN)] * wbc
        rows[e, pl.ds(LN, LN)] = rows[e, pl.ds(LN, LN)] * wbc
      return _
    lax.fori_loop(0, CH // LN, _mul, None, unroll=2)

  def _fire_scatters(bufs):
    rows, _, _, didx, _ = bufs
    for j in range(NSUB):
      pltpu.async_copy(rows.at[pl.ds(j * SUB, SUB)], acc.at[didx.at[j]],
                       sems, add=True)

  def _drain_scatters(bufs):
    rows, _, _, didx, _ = bufs
    for j in range(NSUB):
      pltpu.make_async_copy(rows.at[pl.ds(j * SUB, SUB)],
                            acc.at[didx.at[j]], sems).wait()

  def _zero_acc_slice():
    def _zr(r, _):
      rowsA[r, pl.ds(0, LN)] = zeros
      rowsA[r, pl.ds(LN, LN)] = zeros
      return _
    lax.fori_loop(0, CH, _zr, None, unroll=8)

    for q in range(NZ):
      pltpu.async_copy(rowsA.at[pl.ds(0, CH)],
                       acc.at[pl.ds(rbase + q * CH, CH)], seml)

    @pl.when(jnp.logical_not(is_last))
    def _():
      pltpu.async_copy(rowsA.at[pl.ds(0, RQ - NZ * CH)],
                       acc.at[pl.ds(rbase + NZ * CH, RQ - NZ * CH)], seml)

    @pl.when(is_last)
    def _():
      pltpu.async_copy(rowsA.at[pl.ds(0, RQ_LAST - NZ * CH)],
                       acc.at[pl.ds(rbase + NZ * CH, RQ_LAST - NZ * CH)],
                       seml)

    for q in range(NZ):
      pltpu.make_async_copy(rowsA.at[pl.ds(0, CH)],
                            acc.at[pl.ds(rbase + q * CH, CH)], seml).wait()

    @pl.when(jnp.logical_not(is_last))
    def _():
      pltpu.make_async_copy(rowsA.at[pl.ds(0, RQ - NZ * CH)],
                            acc.at[pl.ds(rbase + NZ * CH, RQ - NZ * CH)],
                            seml).wait()

    @pl.when(is_last)
    def _():
      pltpu.make_async_copy(rowsA.at[pl.ds(0, RQ_LAST - NZ * CH)],
                            acc.at[pl.ds(rbase + NZ * CH, RQ_LAST - NZ * CH)],
                            seml).wait()

  _zero_acc_slice()
  plsc.subcore_barrier()

  for layer in range(LAYERS):
    table = ego0 if layer == 0 else lb.at[c, layer - 1]

    _loads_sw(bufs_a, 0)
    _fire_didx(bufs_a, 0)
    _drain_loads_sw(bufs_a)
    if layer == 0:
      _transform(bufs_a)
    _fire_gathers(bufs_a, table)
    _loads_sw(bufs_b, 1)
    _fire_didx(bufs_b, 1)

    _drain_loads_sw(bufs_b)
    if layer == 0:
      _transform(bufs_b)
    _fire_gathers(bufs_b, table)
    _drain_gathers(bufs_a, table)
    _multiply(bufs_a)
    _drain_didx(bufs_a)
    _fire_scatters(bufs_a)
    _loads_sw(bufs_a, 2)

    def _step(p, q, prep, layer=layer, table=table):
      _drain_scatters(p)
      _fire_didx(p, jnp.minimum(prep, NCHUNK - 1))
      _drain_loads_sw(p)
      if layer == 0:
        _transform(p)
      _fire_gathers(p, table)
      _drain_gathers(q, table)
      _multiply(q)
      _drain_didx(q)
      _fire_scatters(q)
      _loads_sw(q, jnp.minimum(prep + 1, NCHUNK - 1))

    def _pair(i2, _):
      _step(bufs_a, bufs_b, 2 * i2 + 2)
      _step(bufs_b, bufs_a, 2 * i2 + 3)
      return _
    lax.fori_loop(0, NPAIR, _pair, None)

    _drain_scatters(bufs_a)
    _drain_didx(bufs_b)
    _drain_gathers(bufs_b, table)
    _drain_loads_sw(bufs_a)
    plsc.subcore_barrier()

    if layer < LAYERS - 1:
      def _wc(q, _, layer=layer):
        pltpu.sync_copy(acc.at[pl.ds(rbase + q * CH, CH)],
                        rowsA.at[pl.ds(0, CH)])
        pltpu.sync_copy(rowsA.at[pl.ds(0, CH)],
                        lb.at[c, layer, pl.ds(rbase + q * CH, CH)])
        return _
      lax.fori_loop(0, NZ, _wc, None)

      def _wtail(n, layer=layer):
        pltpu.sync_copy(acc.at[pl.ds(rbase + NZ * CH, n)],
                        rowsA.at[pl.ds(0, n)])
        pltpu.sync_copy(rowsA.at[pl.ds(0, n)],
                        lb.at[c, layer, pl.ds(rbase + NZ * CH, n)])

      @pl.when(jnp.logical_not(is_last))
      def _():
        _wtail(RQ - NZ * CH)

      @pl.when(is_last)
      def _():
        _wtail(RQ_LAST - NZ * CH)

      _zero_acc_slice()
      plsc.subcore_barrier()

  third = jnp.full((LN,), 1.0 / LAYERS, F32)
  nq = (NMCH - s + NS - 1) // NS

  def _mean(qi, _):
    row0 = pl.multiple_of((s + qi * NS) * MCH, MCH)
    pltpu.sync_copy(lb.at[c, 0, pl.ds(row0, MCH)], rowsA.at[pl.ds(0, MCH)])
    pltpu.sync_copy(lb.at[c, 1, pl.ds(row0, MCH)], rowsA.at[pl.ds(MCH, MCH)])
    pltpu.sync_copy(acc.at[pl.ds(row0, MCH)], rowsA.at[pl.ds(2 * MCH, MCH)])

    def _avg(r, _):
      for off in (0, LN):
        m = (rowsA[r, pl.ds(off, LN)] + rowsA[MCH + r, pl.ds(off, LN)]
             + rowsA[2 * MCH + r, pl.ds(off, LN)]) * third
        rowsA[3 * MCH + r, pl.ds(off, LN)] = m
      return _
    lax.fori_loop(0, MCH, _avg, None, unroll=4)

    base2 = 2 * row0 + c
    for t in range(VPS):
      oidx[0, pl.ds(t * LN, LN)] = base2 + 2 * (t * LN + lax.iota(I32, LN))
    pltpu.sync_copy(rowsA.at[pl.ds(3 * MCH, MCH)], out.at[oidx.at[0]])
    return _
  lax.fori_loop(0, nq, _mean, None)


def _propagate(ego0, eidx, w):
  mesh = plsc.VectorSubcoreMesh(core_axis_name="c", subcore_axis_name="s",
                                num_cores=NC, num_subcores=NS)
  f = pl.kernel(
      _sc_body,
      out_type=(
          jax.ShapeDtypeStruct((2 * N, H), F32),
          jax.ShapeDtypeStruct((NC, LAYERS - 1, N, H), F32),
      ),
      mesh=mesh,
      compiler_params=pltpu.CompilerParams(use_tc_tiling_on_sc=False),
      scratch_types=[
          pltpu.VMEM_SHARED((N, H), F32),
          pltpu.VMEM((CH, H), F32),
          pltpu.VMEM((CH, H), F32),
          pltpu.VMEM((CH,), I32),
          pltpu.VMEM((CH,), I32),
          pltpu.VMEM((CH,), F32),
          pltpu.VMEM((CH,), F32),
          pltpu.VMEM((NSUB, SUB), I32),
          pltpu.VMEM((NSUB, SUB), I32),
          pltpu.VMEM((1, SUB), I32),
          pltpu.SemaphoreType.DMA,
          pltpu.SemaphoreType.DMA,
          pltpu.SemaphoreType.DMA,
          pltpu.SemaphoreType.DMA,
          pltpu.SemaphoreType.DMA,
      ],
  )
  res, _ = f(ego0, eidx, w)
  return res


def kernel(user_emb, item_emb, edge_index, edge_weight):
  ego0 = jnp.concatenate([user_emb, item_emb], axis=0).reshape(2 * N, H)
  mean = _propagate(ego0, edge_index, edge_weight)
  mean = mean.reshape(N, D)
  return mean[:N_U], mean[N_U:]


# speedup vs baseline: 2.4403x; 2.4403x over previous
import jax
import jax.numpy as jnp
from jax import lax
from jax.experimental import pallas as pl
from jax.experimental.pallas import tpu as pltpu
from jax.experimental.pallas import tpu_sc as plsc

N_U = 25000
N_I = 25000
N = N_U + N_I
E = 800000
D = 64
H = 32
LAYERS = 3
NC = 2
NS = 16
LN = 16

EPT = E // NS
CH = 400
NCHUNK = EPT // CH
NPAIR = (NCHUNK - 1) // 2
SUB = 80
NSUB = CH // SUB
VPS = SUB // LN

RQ = 3128
RQ_LAST = N - (NS - 1) * RQ
NZ = RQ // CH
MCH = 80
NMH = N_U // MCH + 1
NMCH = 2 * NMH
GLAST = N_U - MCH

F32 = jnp.float32
I32 = jnp.int32


def _sc_body(ego0, eidx, wb, outu, outi, lb, acc,
             rowsA, rowsB, srcvA, srcvB, wvA, wvB, didxA, didxB,
             oidx, seml, semd, semgA, semgB, sems):
  c = lax.axis_index("c")
  s = lax.axis_index("s")
  ebase0 = s * EPT
  rbase = s * RQ
  is_last = s == NS - 1
  zeros = jnp.zeros((LN,), F32)

  bufs_a = (rowsA, srcvA, wvA, didxA, semgA)
  bufs_b = (rowsB, srcvB, wvB, didxB, semgB)

  def _loads_sw(bufs, ci):
    _, srcv, wv, _, _ = bufs
    eoff = ebase0 + ci * CH
    pltpu.async_copy(eidx.at[0, pl.ds(eoff, CH)], srcv, seml)
    pltpu.async_copy(wb.at[pl.ds(eoff, CH)], wv, seml)

  def _drain_loads_sw(bufs):
    _, srcv, wv, _, _ = bufs
    pltpu.make_async_copy(eidx.at[0, pl.ds(0, CH)], srcv, seml).wait()
    pltpu.make_async_copy(wb.at[pl.ds(0, CH)], wv, seml).wait()

  def _fire_didx(bufs, ci):
    _, _, _, didx, _ = bufs
    eoff = ebase0 + ci * CH
    for j in range(NSUB):
      pltpu.async_copy(eidx.at[1, pl.ds(eoff + j * SUB, SUB)],
                       didx.at[j], semd)

  def _drain_didx(bufs):
    _, _, _, didx, _ = bufs
    for j in range(NSUB):
      pltpu.make_async_copy(eidx.at[1, pl.ds(0, SUB)], didx.at[j],
                            semd).wait()

  def _transform(bufs):
    _, srcv, _, _, _ = bufs

    def _tf(k, _):
      k16 = pl.multiple_of(k * LN, LN)
      sv = srcv[pl.ds(k16, LN)]
      srcv[pl.ds(k16, LN)] = 2 * sv + c
      return _
    lax.fori_loop(0, CH // LN, _tf, None)

  def _fire_gathers(bufs, table):
    rows, srcv, _, _, semg = bufs
    pltpu.async_copy(table.at[srcv], rows, semg)

  def _drain_gathers(bufs, table):
    rows, srcv, _, _, semg = bufs
    pltpu.make_async_copy(table.at[srcv], rows, semg).wait()

  def _multiply(bufs):
    rows, _, wv, _, _ = bufs

    def _mul(g, _):
      goff = pl.multiple_of(g * LN, LN)
      w16 = wv[pl.ds(goff, LN)]
      for t in range(LN):
        wbc = w16.at[jnp.full((LN,), t, I32)].get(mode="promise_in_bounds")
        e = goff + t
        rows[e, pl.ds(0, LN)] = rows[e, pl.ds(0, LN)] * wbc
        rows[e, pl.ds(LN, LN)] = rows[e, pl.ds(LN, LN)] * wbc
      return _
    lax.fori_loop(0, CH // LN, _mul, None)

  def _fire_scatters(bufs):
    rows, _, _, didx, _ = bufs
    for j in range(NSUB):
      pltpu.async_copy(rows.at[pl.ds(j * SUB, SUB)], acc.at[didx.at[j]],
                       sems, add=True)

  def _drain_scatters(bufs):
    rows, _, _, didx, _ = bufs
    for j in range(NSUB):
      pltpu.make_async_copy(rows.at[pl.ds(j * SUB, SUB)],
                            acc.at[didx.at[j]], sems).wait()

  def _zero_acc_slice():
    def _zr(r, _):
      rowsA[r, pl.ds(0, LN)] = zeros
      rowsA[r, pl.ds(LN, LN)] = zeros
      return _
    lax.fori_loop(0, CH, _zr, None, unroll=8)

    for q in range(NZ):
      pltpu.async_copy(rowsA.at[pl.ds(0, CH)],
                       acc.at[pl.ds(rbase + q * CH, CH)], seml)

    @pl.when(jnp.logical_not(is_last))
    def _():
      pltpu.async_copy(rowsA.at[pl.ds(0, RQ - NZ * CH)],
                       acc.at[pl.ds(rbase + NZ * CH, RQ - NZ * CH)], seml)

    @pl.when(is_last)
    def _():
      pltpu.async_copy(rowsA.at[pl.ds(0, RQ_LAST - NZ * CH)],
                       acc.at[pl.ds(rbase + NZ * CH, RQ_LAST - NZ * CH)],
                       seml)

    for q in range(NZ):
      pltpu.make_async_copy(rowsA.at[pl.ds(0, CH)],
                            acc.at[pl.ds(rbase + q * CH, CH)], seml).wait()

    @pl.when(jnp.logical_not(is_last))
    def _():
      pltpu.make_async_copy(rowsA.at[pl.ds(0, RQ - NZ * CH)],
                            acc.at[pl.ds(rbase + NZ * CH, RQ - NZ * CH)],
                            seml).wait()

    @pl.when(is_last)
    def _():
      pltpu.make_async_copy(rowsA.at[pl.ds(0, RQ_LAST - NZ * CH)],
                            acc.at[pl.ds(rbase + NZ * CH, RQ_LAST - NZ * CH)],
                            seml).wait()

  _zero_acc_slice()
  plsc.subcore_barrier()

  for layer in range(LAYERS):
    table = ego0 if layer == 0 else lb.at[c, layer - 1]

    _loads_sw(bufs_a, 0)
    _fire_didx(bufs_a, 0)
    _drain_loads_sw(bufs_a)
    if layer == 0:
      _transform(bufs_a)
    _fire_gathers(bufs_a, table)
    _loads_sw(bufs_b, 1)
    _fire_didx(bufs_b, 1)

    _drain_loads_sw(bufs_b)
    if layer == 0:
      _transform(bufs_b)
    _fire_gathers(bufs_b, table)
    _drain_gathers(bufs_a, table)
    _multiply(bufs_a)
    _drain_didx(bufs_a)
    _fire_scatters(bufs_a)
    _loads_sw(bufs_a, 2)

    def _step(p, q, prep, layer=layer, table=table):
      _drain_scatters(p)
      _fire_didx(p, jnp.minimum(prep, NCHUNK - 1))
      _drain_loads_sw(p)
      if layer == 0:
        _transform(p)
      _fire_gathers(p, table)
      _drain_gathers(q, table)
      _multiply(q)
      _drain_didx(q)
      _fire_scatters(q)
      _loads_sw(q, jnp.minimum(prep + 1, NCHUNK - 1))

    def _pair(i2, _):
      _step(bufs_a, bufs_b, 2 * i2 + 2)
      _step(bufs_b, bufs_a, 2 * i2 + 3)
      return _
    lax.fori_loop(0, NPAIR, _pair, None)

    _drain_scatters(bufs_a)
    _drain_didx(bufs_b)
    _drain_gathers(bufs_b, table)
    _drain_loads_sw(bufs_a)
    plsc.subcore_barrier()

    if layer < LAYERS - 1:
      def _wc(q, _, layer=layer):
        pltpu.sync_copy(acc.at[pl.ds(rbase + q * CH, CH)],
                        rowsA.at[pl.ds(0, CH)])
        pltpu.sync_copy(rowsA.at[pl.ds(0, CH)],
                        lb.at[c, layer, pl.ds(rbase + q * CH, CH)])
        return _
      lax.fori_loop(0, NZ, _wc, None)

      def _wtail(n, layer=layer):
        pltpu.sync_copy(acc.at[pl.ds(rbase + NZ * CH, n)],
                        rowsA.at[pl.ds(0, n)])
        pltpu.sync_copy(rowsA.at[pl.ds(0, n)],
                        lb.at[c, layer, pl.ds(rbase + NZ * CH, n)])

      @pl.when(jnp.logical_not(is_last))
      def _():
        _wtail(RQ - NZ * CH)

      @pl.when(is_last)
      def _():
        _wtail(RQ_LAST - NZ * CH)

      _zero_acc_slice()
      plsc.subcore_barrier()

  third = jnp.full((LN,), 1.0 / LAYERS, F32)
  nq = (NMCH - s + NS - 1) // NS

  def _mean(qi, _):
    q = s + qi * NS
    is_user = q < NMH
    lr = jnp.minimum(jnp.where(is_user, q, q - NMH) * MCH, GLAST)
    g0 = jnp.where(is_user, lr, N_U + lr)
    pltpu.sync_copy(lb.at[c, 0, pl.ds(g0, MCH)], rowsA.at[pl.ds(0, MCH)])
    pltpu.sync_copy(lb.at[c, 1, pl.ds(g0, MCH)], rowsA.at[pl.ds(MCH, MCH)])
    pltpu.sync_copy(acc.at[pl.ds(g0, MCH)], rowsA.at[pl.ds(2 * MCH, MCH)])

    def _avg(r, _):
      for off in (0, LN):
        m = (rowsA[r, pl.ds(off, LN)] + rowsA[MCH + r, pl.ds(off, LN)]
             + rowsA[2 * MCH + r, pl.ds(off, LN)]) * third
        rowsA[3 * MCH + r, pl.ds(off, LN)] = m
      return _
    lax.fori_loop(0, MCH, _avg, None, unroll=4)

    base2 = 2 * lr + c
    for t in range(VPS):
      oidx[0, pl.ds(t * LN, LN)] = base2 + 2 * (t * LN + lax.iota(I32, LN))

    @pl.when(is_user)
    def _():
      pltpu.sync_copy(rowsA.at[pl.ds(3 * MCH, MCH)], outu.at[oidx.at[0]])

    @pl.when(jnp.logical_not(is_user))
    def _():
      pltpu.sync_copy(rowsA.at[pl.ds(3 * MCH, MCH)], outi.at[oidx.at[0]])
    return _
  lax.fori_loop(0, nq, _mean, None)


def _propagate(ego0, eidx, w):
  mesh = plsc.VectorSubcoreMesh(core_axis_name="c", subcore_axis_name="s",
                                num_cores=NC, num_subcores=NS)
  f = pl.kernel(
      _sc_body,
      out_type=(
          jax.ShapeDtypeStruct((2 * N_U, H), F32),
          jax.ShapeDtypeStruct((2 * N_I, H), F32),
          jax.ShapeDtypeStruct((NC, LAYERS - 1, N, H), F32),
      ),
      mesh=mesh,
      compiler_params=pltpu.CompilerParams(use_tc_tiling_on_sc=False),
      scratch_types=[
          pltpu.VMEM_SHARED((N, H), F32),
          pltpu.VMEM((CH, H), F32),
          pltpu.VMEM((CH, H), F32),
          pltpu.VMEM((CH,), I32),
          pltpu.VMEM((CH,), I32),
          pltpu.VMEM((CH,), F32),
          pltpu.VMEM((CH,), F32),
          pltpu.VMEM((NSUB, SUB), I32),
          pltpu.VMEM((NSUB, SUB), I32),
          pltpu.VMEM((1, SUB), I32),
          pltpu.SemaphoreType.DMA,
          pltpu.SemaphoreType.DMA,
          pltpu.SemaphoreType.DMA,
          pltpu.SemaphoreType.DMA,
          pltpu.SemaphoreType.DMA,
      ],
  )
  ou, oi, _ = f(ego0, eidx, w)
  return ou, oi


def kernel(user_emb, item_emb, edge_index, edge_weight):
  ego0 = jnp.concatenate([user_emb, item_emb], axis=0).reshape(2 * N, H)
  ou, oi = _propagate(ego0, edge_index, edge_weight)
  return ou.reshape(N_U, D), oi.reshape(N_I, D)
